# baseline (speedup 1.0000x reference)
import functools

import jax
import jax.numpy as jnp
from jax import lax
from jax.experimental import pallas as pl
from jax.experimental.pallas import tpu as pltpu
from jax.experimental.pallas import tpu_sc as plsc

N = 10000
E = 160000
EP = 163840
BN = 1000
GRID = N // BN
F32 = jnp.float32


def _lrelu(x):
    return jnp.where(x >= 0, x, 0.2 * x)


def _stage_a_body(x_ref, w1_ref, attf_ref, h1_ref, a1_ref):
    h = jnp.dot(x_ref[...], w1_ref[...], preferred_element_type=F32)
    h1_ref[...] = h
    asrc = (h * attf_ref[0][None, :]).reshape(BN, 4, 256).sum(axis=-1)
    adst = (h * attf_ref[1][None, :]).reshape(BN, 4, 256).sum(axis=-1)
    a1_ref[...] = jnp.concatenate([asrc, adst], axis=1)


def _stage_a(x, W1, attf):
    return pl.pallas_call(
        _stage_a_body,
        grid=(GRID,),
        in_specs=[
            pl.BlockSpec((BN, 256), lambda i: (i, 0)),
            pl.BlockSpec((256, 1024), lambda i: (0, 0)),
            pl.BlockSpec((2, 1024), lambda i: (0, 0)),
        ],
        out_specs=[
            pl.BlockSpec((BN, 1024), lambda i: (i, 0)),
            pl.BlockSpec((BN, 8), lambda i: (i, 0)),
        ],
        out_shape=[
            jax.ShapeDtypeStruct((N, 1024), F32),
            jax.ShapeDtypeStruct((N, 8), F32),
        ],
    )(x, W1, attf)


def _stage_d_body(acc_ref, den_ref, a1_ref, h1s_ref, b1r_ref, w2r_ref,
                  att2_ref, b2_ref, h2_ref, a2_ref):
    den = den_ref[0] + den_ref[1]
    h2 = jnp.zeros((BN, 128), F32)
    for s in range(8):
        hh = s // 2
        wself = jnp.exp(_lrelu(a1_ref[:, hh:hh + 1] + a1_ref[:, 4 + hh:5 + hh]))
        t = acc_ref[s, 0] + acc_ref[s, 1] + wself * h1s_ref[:, s, :]
        t = t / (den[:, hh:hh + 1] + wself)
        t = t + b1r_ref[s][None, :]
        e = jnp.where(t > 0, t, jnp.exp(t) - 1.0)
        h2 = h2 + jnp.dot(e, w2r_ref[s], preferred_element_type=F32)
    h2 = h2 + b2_ref[...]
    h2_ref[...] = h2
    a2s = jnp.sum(h2 * att2_ref[0][None, :], axis=-1, keepdims=True)
    a2d = jnp.sum(h2 * att2_ref[1][None, :], axis=-1, keepdims=True)
    a2_ref[...] = jnp.concatenate([a2s, a2d], axis=1)


def _stage_d(acc1, den1t, a1, h1s, b1r, W2r, att2, b2):
    return pl.pallas_call(
        _stage_d_body,
        grid=(GRID,),
        in_specs=[
            pl.BlockSpec((8, 2, BN, 128), lambda i: (0, 0, i, 0)),
            pl.BlockSpec((2, BN, 4), lambda i: (0, i, 0)),
            pl.BlockSpec((BN, 8), lambda i: (i, 0)),
            pl.BlockSpec((BN, 8, 128), lambda i: (i, 0, 0)),
            pl.BlockSpec((8, 128), lambda i: (0, 0)),
            pl.BlockSpec((8, 128, 128), lambda i: (0, 0, 0)),
            pl.BlockSpec((2, 128), lambda i: (0, 0)),
            pl.BlockSpec((1, 128), lambda i: (0, 0)),
        ],
        out_specs=[
            pl.BlockSpec((BN, 128), lambda i: (i, 0)),
            pl.BlockSpec((BN, 2), lambda i: (i, 0)),
        ],
        out_shape=[
            jax.ShapeDtypeStruct((N, 128), F32),
            jax.ShapeDtypeStruct((N, 2), F32),
        ],
    )(acc1, den1t, a1, h1s, b1r, W2r, att2, b2)


def _stage_f_body(acc2_ref, den2_ref, a2_ref, h2_ref, b2_ref, batch_ref,
                  out_ref, cnt_ref):
    i = pl.program_id(0)
    wself = jnp.exp(_lrelu(a2_ref[:, 0:1] + a2_ref[:, 1:2]))
    den = den2_ref[:, 0:1] + den2_ref[:, 1:2] + wself
    t = (acc2_ref[0] + acc2_ref[1] + wself * h2_ref[...]) / den + b2_ref[...]
    gids = lax.broadcasted_iota(jnp.int32, (BN, 64), 1)
    oh = (batch_ref[...] == gids).astype(F32)
    pool = lax.dot_general(oh, t, (((0,), (0,)), ((), ())),
                           preferred_element_type=F32)
    c128 = lax.dot_general(oh, jnp.ones((BN, 128), F32),
                           (((0,), (0,)), ((), ())),
                           preferred_element_type=F32)

    @pl.when(i == 0)
    def _():
        out_ref[...] = jnp.zeros_like(out_ref)
        cnt_ref[...] = jnp.zeros_like(cnt_ref)

    out_ref[...] += pool
    cnt_ref[...] += c128

    @pl.when(i == GRID - 1)
    def _():
        out_ref[...] = out_ref[...] / jnp.maximum(cnt_ref[...], 1.0)


def _stage_f(acc2, den2t, a2, h2, b2, batch2d):
    return pl.pallas_call(
        _stage_f_body,
        grid=(GRID,),
        in_specs=[
            pl.BlockSpec((2, BN, 128), lambda i: (0, i, 0)),
            pl.BlockSpec((BN, 2), lambda i: (i, 0)),
            pl.BlockSpec((BN, 2), lambda i: (i, 0)),
            pl.BlockSpec((BN, 128), lambda i: (i, 0)),
            pl.BlockSpec((1, 128), lambda i: (0, 0)),
            pl.BlockSpec((BN, 1), lambda i: (i, 0)),
        ],
        out_specs=pl.BlockSpec((64, 128), lambda i: (0, 0)),
        out_shape=jax.ShapeDtypeStruct((64, 128), F32),
        scratch_shapes=[pltpu.VMEM((64, 128), F32)],
    )(acc2, den2t, a2, h2, b2, batch2d)


NC = 2
NS = 16
NW = NC * NS
EW = EP // NW
NB = EW // 128
B0 = 48
B1 = 80 - B0
WIN = 24
NW0 = (B0 + WIN - 1) // WIN
NW1 = (B1 + WIN - 1) // WIN
EP2 = EP + WIN * 128
NPAD = 10240
ACC_R = 10112
STR = ACC_R // NS
DSTR = NPAD // NS
CHUNKS = [(0, 128), (128, 128), (256, 128), (384, 128), (512, 120)]

_sc_mesh = None


def _mesh():
    global _sc_mesh
    if _sc_mesh is None:
        _sc_mesh = plsc.VectorSubcoreMesh(
            core_axis_name="c", subcore_axis_name="s",
            num_cores=NC, num_subcores=NS)
    return _sc_mesh


def _splat(val):
    return lax.iota(jnp.int32, 16) * 0 + val


def _iota16():
    return lax.iota(jnp.int32, 16)


def _zero_vec(ref, n):
    z = jnp.zeros((16,), F32)
    for i in range(0, n, 16):
        ref[pl.ds(i, 16)] = z


def _zero_rows(rows_v, n):
    z = jnp.zeros((16,), F32)

    def body(r, _):
        for c in range(8):
            rows_v[r, pl.ds(c * 16, 16)] = z
        return 0
    lax.fori_loop(0, n, body, 0)


def _zero_acc_stripe(acc_sh, rows_v, sid):
    for off, ln in CHUNKS:
        pltpu.sync_copy(rows_v.at[pl.ds(0, ln)],
                        acc_sh.at[pl.ds(sid * STR + off, ln)])


def _dump_acc_stripe(acc_sh, rows_v, out_at, sid):
    for off, ln in CHUNKS:
        pltpu.sync_copy(acc_sh.at[pl.ds(sid * STR + off, ln)],
                        rows_v.at[pl.ds(0, ln)])
        pltpu.sync_copy(rows_v.at[pl.ds(0, ln)],
                        out_at.at[pl.ds(sid * STR + off, ln)])


def _scale_rows(rows_v, rbase, w_v, wbase):
    def body(r, _):
        wsp = plsc.load_gather(w_v, [_splat(wbase + r)])
        for c in range(8):
            rows_v[rbase + r, pl.ds(c * 16, 16)] = (
                rows_v[rbase + r, pl.ds(c * 16, 16)] * wsp)
        return 0
    lax.fori_loop(0, 128, body, 0)


def _fire_alpha(stab, dtab, src_v, dst_v, j, asb, adb, sem):
    pltpu.async_copy(stab.at[src_v.at[pl.ds(j * 128, 128)]], asb, sem)
    pltpu.async_copy(dtab.at[dst_v.at[j]], adb, sem)


def _wait_alpha(stab, dtab, src_v, dst_v, asb, adb, sem):
    pltpu.make_async_copy(stab.at[src_v.at[pl.ds(0, 128)]], asb, sem).wait()
    pltpu.make_async_copy(dtab.at[dst_v.at[0]], adb, sem).wait()


def _w_block(j, asb, adb, w_v, ebase):
    for c in range(8):
        a = asb[pl.ds(c * 16, 16)] + adb[pl.ds(c * 16, 16)]
        a = jnp.where(a >= 0, a, a * 0.2)
        w16 = jnp.exp(a)
        gid = _iota16() + (ebase + j * 128 + c * 16)
        w_v[pl.ds(j * 128 + c * 16, 16)] = jnp.where(gid < E, w16, 0.0)


def _w_pass(nb, stab, dtab, src_v, dst_v, w_v, den_sh, ebase, do_den,
            asbA, adbA, asbB, adbB, semA, semB):
    _fire_alpha(stab, dtab, src_v, dst_v, 0, asbA, adbA, semA)

    def body(t, _):
        j0 = 2 * t
        j1 = 2 * t + 1
        _fire_alpha(stab, dtab, src_v, dst_v, j1, asbB, adbB, semB)
        _wait_alpha(stab, dtab, src_v, dst_v, asbA, adbA, semA)
        _w_block(j0, asbA, adbA, w_v, ebase)
        if do_den:
            pltpu.sync_copy(w_v.at[pl.ds(j0 * 128, 128)],
                            den_sh.at[dst_v.at[j0]], add=True)

        @pl.when(t < nb // 2 - 1)
        def _():
            _fire_alpha(stab, dtab, src_v, dst_v, j0 + 2, asbA, adbA, semA)
        _wait_alpha(stab, dtab, src_v, dst_v, asbB, adbB, semB)
        _w_block(j1, asbB, adbB, w_v, ebase)
        if do_den:
            pltpu.sync_copy(w_v.at[pl.ds(j1 * 128, 128)],
                            den_sh.at[dst_v.at[j1]], add=True)
        return 0
    lax.fori_loop(0, nb // 2, body, 0)


def _feat_pass(nb, rowtab_hbm, src_v, dst_v, w_v, acc_sh, rows_v, semA, semB):
    def fire(j, rbase, sem):
        pltpu.async_copy(rowtab_hbm.at[src_v.at[pl.ds(j * 128, 128)]],
                         rows_v.at[pl.ds(rbase, 128)], sem)

    def wait(rbase, sem):
        pltpu.make_async_copy(rowtab_hbm.at[src_v.at[pl.ds(0, 128)]],
                              rows_v.at[pl.ds(rbase, 128)], sem).wait()

    fire(0, 0, semA)

    def body(t, _):
        j0 = 2 * t
        j1 = 2 * t + 1
        fire(j1, 128, semB)
        wait(0, semA)
        _scale_rows(rows_v, 0, w_v, j0 * 128)
        pltpu.sync_copy(rows_v.at[pl.ds(0, 128)],
                        acc_sh.at[dst_v.at[j0]], add=True)

        @pl.when(t < nb // 2 - 1)
        def _():
            fire(j0 + 2, 0, semA)
        wait(128, semB)
        _scale_rows(rows_v, 128, w_v, j1 * 128)
        pltpu.sync_copy(rows_v.at[pl.ds(128, 128)],
                        acc_sh.at[dst_v.at[j1]], add=True)
        return 0
    lax.fori_loop(0, nb // 2, body, 0)


_SC1_SCRATCH = [pltpu.VMEM((WIN * 128,), jnp.int32),
                pltpu.VMEM((WIN, 128), jnp.int32),
                pltpu.VMEM((WIN * 128,), F32),
                pltpu.VMEM((256, 128), F32),
                pltpu.VMEM((128,), F32),
                pltpu.VMEM((128,), F32),
                pltpu.VMEM((128,), F32),
                pltpu.VMEM((128,), F32),
                pltpu.VMEM_SHARED((ACC_R, 128), F32),
                pltpu.VMEM_SHARED((NPAD,), F32),
                pltpu.SemaphoreType.DMA,
                pltpu.SemaphoreType.DMA]


def _den_zero_stripe(den_sh, asbA, sid):
    z = jnp.zeros((16,), F32)
    for i in range(0, 128, 16):
        asbA[pl.ds(i, 16)] = z
    for kk in range(5):
        pltpu.sync_copy(asbA, den_sh.at[pl.ds(sid * DSTR + kk * 128, 128)])


def _den_dump_stripe(den_sh, asbA, den_out, base, sid):
    for kk in range(5):
        pltpu.sync_copy(den_sh.at[pl.ds(sid * DSTR + kk * 128, 128)], asbA)
        pltpu.sync_copy(asbA, den_out.at[pl.ds(base + sid * DSTR
                                               + kk * 128, 128)])


def _edge_windows(s_idx, scale8, do_den, nwin, nb, bbase, stab, dtab,
                  rowtab_hbm, src_hbm, dst2d_hbm, src_v, dst_v, w_v,
                  rows_v, acc_sh, den_sh, asbA, adbA, asbB, adbB,
                  semA, semB):
    def win_body(win, _):
        wb = bbase + win * WIN
        wcount = jnp.minimum(nb - win * WIN, WIN)
        pltpu.sync_copy(src_hbm.at[pl.ds(wb * 128, WIN * 128)], src_v)
        pltpu.sync_copy(dst2d_hbm.at[pl.ds(wb, WIN)], dst_v)
        _w_pass(wcount, stab, dtab, src_v, dst_v, w_v, den_sh,
                wb * 128, do_den, asbA, adbA, asbB, adbB, semA, semB)
        if scale8 is not None:
            def sc_idx(j, _):
                for c in range(8):
                    off = j * 128 + c * 16
                    src_v[pl.ds(off, 16)] = (src_v[pl.ds(off, 16)] * 8
                                             + scale8)
                return 0
            lax.fori_loop(0, wcount, sc_idx, 0)
        _feat_pass(wcount, rowtab_hbm, src_v, dst_v, w_v, acc_sh, rows_v,
                   semA, semB)
        return 0
    lax.fori_loop(0, nwin, win_body, 0)


def _sc_layer1(src, dst2d, atabs, h1r):
    @functools.partial(
        pl.kernel, mesh=_mesh(),
        out_type=[jax.ShapeDtypeStruct((8, NC, ACC_R, 128), F32),
                  jax.ShapeDtypeStruct((NC * 4 * NPAD,), F32)],
        scratch_types=_SC1_SCRATCH,
        compiler_params=pltpu.CompilerParams(needs_layout_passes=False))
    def k(src_hbm, dst2d_hbm, as0, as1, as2, as3, ad0, ad1, ad2, ad3,
          h1r_hbm, acc_out, den_out,
          src_v, dst_v, w_v, rows_v, asbA, adbA, asbB, adbB,
          acc_sh, den_sh, semA, semB):
        cid = lax.axis_index("c")
        sid = lax.axis_index("s")
        nb = lax.select(cid == 0, B0, B1)
        nwin = lax.select(cid == 0, NW0, NW1)
        bbase = lax.select(cid == 0, sid * B0, NS * B0 + sid * B1)
        stabs = [as0, as1, as2, as3]
        dtabs = [ad0, ad1, ad2, ad3]

        _den_zero_stripe(den_sh, asbA, sid)

        for s in range(8):
            h = s // 2
            _zero_rows(rows_v, 128)
            _zero_acc_stripe(acc_sh, rows_v, sid)
            plsc.subcore_barrier()

            _edge_windows(s, s, s % 2 == 0, nwin, nb, bbase,
                          stabs[h], dtabs[h], h1r_hbm, src_hbm, dst2d_hbm,
                          src_v, dst_v, w_v, rows_v, acc_sh, den_sh,
                          asbA, adbA, asbB, adbB, semA, semB)
            plsc.subcore_barrier()

            _dump_acc_stripe(acc_sh, rows_v, acc_out.at[s, cid], sid)
            if s % 2 == 1:
                _den_dump_stripe(den_sh, asbA, den_out,
                                 cid * 4 * NPAD + h * NPAD, sid)
                _den_zero_stripe(den_sh, asbA, sid)

    return k(src, dst2d, *atabs, h1r)


def _sc_layer2(src, dst2d, stab, dtab, h2):
    @functools.partial(
        pl.kernel, mesh=_mesh(),
        out_type=[jax.ShapeDtypeStruct((NC, ACC_R, 128), F32),
                  jax.ShapeDtypeStruct((NC * NPAD,), F32)],
        scratch_types=_SC1_SCRATCH,
        compiler_params=pltpu.CompilerParams(needs_layout_passes=False))
    def k(src_hbm, dst2d_hbm, stab_hbm, dtab_hbm, h2_hbm, acc_out, den_out,
          src_v, dst_v, w_v, rows_v, asbA, adbA, asbB, adbB,
          acc_sh, den_sh, semA, semB):
        cid = lax.axis_index("c")
        sid = lax.axis_index("s")
        nb = lax.select(cid == 0, B0, B1)
        nwin = lax.select(cid == 0, NW0, NW1)
        bbase = lax.select(cid == 0, sid * B0, NS * B0 + sid * B1)

        _den_zero_stripe(den_sh, asbA, sid)
        _zero_rows(rows_v, 128)
        _zero_acc_stripe(acc_sh, rows_v, sid)
        plsc.subcore_barrier()

        _edge_windows(0, None, True, nwin, nb, bbase,
                      stab_hbm, dtab_hbm, h2_hbm, src_hbm, dst2d_hbm,
                      src_v, dst_v, w_v, rows_v, acc_sh, den_sh,
                      asbA, adbA, asbB, adbB, semA, semB)
        plsc.subcore_barrier()

        _den_dump_stripe(den_sh, asbA, den_out, cid * NPAD, sid)
        _dump_acc_stripe(acc_sh, rows_v, acc_out.at[cid], sid)

    return k(src, dst2d, stab, dtab, h2)


def kernel(x, edge_index, batch, W1, att_src1, att_dst1, bias1,
           W2, att_src2, att_dst2, bias2):
    srcp = jnp.concatenate(
        [edge_index[0], jnp.zeros((EP2 - E,), jnp.int32)])
    dstp = jnp.concatenate(
        [edge_index[1], jnp.zeros((EP2 - E,), jnp.int32)])

    attf = jnp.stack([att_src1.reshape(-1), att_dst1.reshape(-1)])
    h1, a1 = _stage_a(x, W1, attf)

    h1r = h1.reshape(N * 8, 128)
    h1s = h1.reshape(N, 8, 128)
    dst2d = dstp.reshape(EP2 // 128, 128)
    a1p = jnp.pad(a1, ((0, NPAD - N), (0, 0)))
    atabs = [a1p[:, i] for i in range(8)]
    acc1, den1f = _sc_layer1(srcp, dst2d, atabs, h1r)
    den1t = jnp.transpose(den1f.reshape(NC, 4, NPAD)[:, :, :N], (0, 2, 1))

    b1r = bias1.reshape(8, 128)
    W2r = W2.reshape(8, 128, 128)
    att2 = jnp.concatenate([att_src2, att_dst2], axis=0)
    b2 = bias2.reshape(1, 128)
    h2, a2 = _stage_d(acc1, den1t, a1, h1s, b1r, W2r, att2, b2)

    a2p = jnp.pad(a2, ((0, NPAD - N), (0, 0)))
    acc2, den2f = _sc_layer2(srcp, dst2d, a2p[:, 0], a2p[:, 1], h2)
    den2t = den2f.reshape(NC, NPAD)[:, :N].T

    batch2d = batch.reshape(N, 1)
    return _stage_f(acc2, den2t, a2, h2, b2, batch2d)


# speedup vs baseline: 1.0426x; 1.0426x over previous
import functools

import jax
import jax.numpy as jnp
from jax import lax
from jax.experimental import pallas as pl
from jax.experimental.pallas import tpu as pltpu
from jax.experimental.pallas import tpu_sc as plsc

N = 10000
E = 160000
EP = 163840
BN = 1000
GRID = N // BN
F32 = jnp.float32


def _lrelu(x):
    return jnp.where(x >= 0, x, 0.2 * x)


def _stage_a_body(x_ref, w1_ref, attf_ref, h1_ref, a1_ref):
    h = jnp.dot(x_ref[...], w1_ref[...], preferred_element_type=F32)
    h1_ref[...] = h
    asrc = (h * attf_ref[0][None, :]).reshape(BN, 4, 256).sum(axis=-1)
    adst = (h * attf_ref[1][None, :]).reshape(BN, 4, 256).sum(axis=-1)
    a1_ref[...] = jnp.concatenate([asrc, adst], axis=1)


def _stage_a(x, W1, attf):
    return pl.pallas_call(
        _stage_a_body,
        grid=(GRID,),
        in_specs=[
            pl.BlockSpec((BN, 256), lambda i: (i, 0)),
            pl.BlockSpec((256, 1024), lambda i: (0, 0)),
            pl.BlockSpec((2, 1024), lambda i: (0, 0)),
        ],
        out_specs=[
            pl.BlockSpec((BN, 1024), lambda i: (i, 0)),
            pl.BlockSpec((BN, 8), lambda i: (i, 0)),
        ],
        out_shape=[
            jax.ShapeDtypeStruct((N, 1024), F32),
            jax.ShapeDtypeStruct((N, 8), F32),
        ],
    )(x, W1, attf)


def _stage_d_body(acc_ref, den_ref, a1_ref, h1s_ref, b1r_ref, w2r_ref,
                  att2_ref, b2_ref, h2_ref, a2_ref):
    den = den_ref[0] + den_ref[1]
    h2 = jnp.zeros((BN, 128), F32)
    for s in range(8):
        hh = s // 2
        wself = jnp.exp(_lrelu(a1_ref[:, hh:hh + 1] + a1_ref[:, 4 + hh:5 + hh]))
        t = acc_ref[s, 0] + acc_ref[s, 1] + wself * h1s_ref[:, s, :]
        t = t / (den[:, hh:hh + 1] + wself)
        t = t + b1r_ref[s][None, :]
        e = jnp.where(t > 0, t, jnp.exp(t) - 1.0)
        h2 = h2 + jnp.dot(e, w2r_ref[s], preferred_element_type=F32)
    h2 = h2 + b2_ref[...]
    h2_ref[...] = h2
    a2s = jnp.sum(h2 * att2_ref[0][None, :], axis=-1, keepdims=True)
    a2d = jnp.sum(h2 * att2_ref[1][None, :], axis=-1, keepdims=True)
    a2_ref[...] = jnp.concatenate([a2s, a2d], axis=1)


def _stage_d(acc1, den1t, a1, h1s, b1r, W2r, att2, b2):
    return pl.pallas_call(
        _stage_d_body,
        grid=(GRID,),
        in_specs=[
            pl.BlockSpec((8, 2, BN, 128), lambda i: (0, 0, i, 0)),
            pl.BlockSpec((2, BN, 4), lambda i: (0, i, 0)),
            pl.BlockSpec((BN, 8), lambda i: (i, 0)),
            pl.BlockSpec((BN, 8, 128), lambda i: (i, 0, 0)),
            pl.BlockSpec((8, 128), lambda i: (0, 0)),
            pl.BlockSpec((8, 128, 128), lambda i: (0, 0, 0)),
            pl.BlockSpec((2, 128), lambda i: (0, 0)),
            pl.BlockSpec((1, 128), lambda i: (0, 0)),
        ],
        out_specs=[
            pl.BlockSpec((BN, 128), lambda i: (i, 0)),
            pl.BlockSpec((BN, 2), lambda i: (i, 0)),
        ],
        out_shape=[
            jax.ShapeDtypeStruct((N, 128), F32),
            jax.ShapeDtypeStruct((N, 2), F32),
        ],
    )(acc1, den1t, a1, h1s, b1r, W2r, att2, b2)


def _stage_f_body(acc2_ref, den2_ref, a2_ref, h2_ref, b2_ref, batch_ref,
                  out_ref, cnt_ref):
    i = pl.program_id(0)
    wself = jnp.exp(_lrelu(a2_ref[:, 0:1] + a2_ref[:, 1:2]))
    den = den2_ref[:, 0:1] + den2_ref[:, 1:2] + wself
    t = (acc2_ref[0] + acc2_ref[1] + wself * h2_ref[...]) / den + b2_ref[...]
    gids = lax.broadcasted_iota(jnp.int32, (BN, 64), 1)
    oh = (batch_ref[...] == gids).astype(F32)
    pool = lax.dot_general(oh, t, (((0,), (0,)), ((), ())),
                           preferred_element_type=F32)
    c128 = lax.dot_general(oh, jnp.ones((BN, 128), F32),
                           (((0,), (0,)), ((), ())),
                           preferred_element_type=F32)

    @pl.when(i == 0)
    def _():
        out_ref[...] = jnp.zeros_like(out_ref)
        cnt_ref[...] = jnp.zeros_like(cnt_ref)

    out_ref[...] += pool
    cnt_ref[...] += c128

    @pl.when(i == GRID - 1)
    def _():
        out_ref[...] = out_ref[...] / jnp.maximum(cnt_ref[...], 1.0)


def _stage_f(acc2, den2t, a2, h2, b2, batch2d):
    return pl.pallas_call(
        _stage_f_body,
        grid=(GRID,),
        in_specs=[
            pl.BlockSpec((2, BN, 128), lambda i: (0, i, 0)),
            pl.BlockSpec((BN, 2), lambda i: (i, 0)),
            pl.BlockSpec((BN, 2), lambda i: (i, 0)),
            pl.BlockSpec((BN, 128), lambda i: (i, 0)),
            pl.BlockSpec((1, 128), lambda i: (0, 0)),
            pl.BlockSpec((BN, 1), lambda i: (i, 0)),
        ],
        out_specs=pl.BlockSpec((64, 128), lambda i: (0, 0)),
        out_shape=jax.ShapeDtypeStruct((64, 128), F32),
        scratch_shapes=[pltpu.VMEM((64, 128), F32)],
    )(acc2, den2t, a2, h2, b2, batch2d)


NC = 2
NS = 16
NW = NC * NS
EW = EP // NW
NB = EW // 128
B0 = 64
B1 = 80 - B0
WIN = 24
NW0 = (B0 + WIN - 1) // WIN
NW1 = (B1 + WIN - 1) // WIN
EP2 = EP + WIN * 128
NPAD = 10240
ACC_R = 10112
STR = ACC_R // NS
DSTR = NPAD // NS
CHUNKS = [(0, 128), (128, 128), (256, 128), (384, 128), (512, 120)]

_sc_mesh = None


def _mesh():
    global _sc_mesh
    if _sc_mesh is None:
        _sc_mesh = plsc.VectorSubcoreMesh(
            core_axis_name="c", subcore_axis_name="s",
            num_cores=NC, num_subcores=NS)
    return _sc_mesh


def _splat(val):
    return lax.iota(jnp.int32, 16) * 0 + val


def _iota16():
    return lax.iota(jnp.int32, 16)


def _zero_vec(ref, n):
    z = jnp.zeros((16,), F32)
    for i in range(0, n, 16):
        ref[pl.ds(i, 16)] = z


def _zero_rows(rows_v, n):
    z = jnp.zeros((16,), F32)

    def body(r, _):
        for c in range(8):
            rows_v[r, pl.ds(c * 16, 16)] = z
        return 0
    lax.fori_loop(0, n, body, 0)


def _zero_acc_stripe(acc_sh, rows_v, sid):
    for off, ln in CHUNKS:
        pltpu.sync_copy(rows_v.at[pl.ds(0, ln)],
                        acc_sh.at[pl.ds(sid * STR + off, ln)])


def _dump_acc_stripe(acc_sh, rows_v, out_at, sid):
    for off, ln in CHUNKS:
        pltpu.sync_copy(acc_sh.at[pl.ds(sid * STR + off, ln)],
                        rows_v.at[pl.ds(0, ln)])
        pltpu.sync_copy(rows_v.at[pl.ds(0, ln)],
                        out_at.at[pl.ds(sid * STR + off, ln)])


def _scale_rows(rows_v, rbase, w_v, wbase):
    def body(r, _):
        wsp = plsc.load_gather(w_v, [_splat(wbase + r)])
        for c in range(8):
            rows_v[rbase + r, pl.ds(c * 16, 16)] = (
                rows_v[rbase + r, pl.ds(c * 16, 16)] * wsp)
        return 0
    lax.fori_loop(0, 128, body, 0)


def _fire_alpha(stab, dtab, src_v, dst_v, j, asb, adb, sem):
    pltpu.async_copy(stab.at[src_v.at[pl.ds(j * 128, 128)]], asb, sem)
    pltpu.async_copy(dtab.at[dst_v.at[j]], adb, sem)


def _wait_alpha(stab, dtab, src_v, dst_v, asb, adb, sem):
    pltpu.make_async_copy(stab.at[src_v.at[pl.ds(0, 128)]], asb, sem).wait()
    pltpu.make_async_copy(dtab.at[dst_v.at[0]], adb, sem).wait()


def _w_block(j, asb, adb, w_v, ebase):
    for c in range(8):
        a = asb[pl.ds(c * 16, 16)] + adb[pl.ds(c * 16, 16)]
        a = jnp.where(a >= 0, a, a * 0.2)
        w16 = jnp.exp(a)
        gid = _iota16() + (ebase + j * 128 + c * 16)
        w_v[pl.ds(j * 128 + c * 16, 16)] = jnp.where(gid < E, w16, 0.0)


def _w_pass(nb, stab, dtab, src_v, dst_v, w_v, den_sh, ebase, do_den,
            asbA, adbA, asbB, adbB, semA, semB):
    _fire_alpha(stab, dtab, src_v, dst_v, 0, asbA, adbA, semA)

    def body(t, _):
        j0 = 2 * t
        j1 = 2 * t + 1
        _fire_alpha(stab, dtab, src_v, dst_v, j1, asbB, adbB, semB)
        _wait_alpha(stab, dtab, src_v, dst_v, asbA, adbA, semA)
        _w_block(j0, asbA, adbA, w_v, ebase)
        if do_den:
            pltpu.sync_copy(w_v.at[pl.ds(j0 * 128, 128)],
                            den_sh.at[dst_v.at[j0]], add=True)

        @pl.when(t < nb // 2 - 1)
        def _():
            _fire_alpha(stab, dtab, src_v, dst_v, j0 + 2, asbA, adbA, semA)
        _wait_alpha(stab, dtab, src_v, dst_v, asbB, adbB, semB)
        _w_block(j1, asbB, adbB, w_v, ebase)
        if do_den:
            pltpu.sync_copy(w_v.at[pl.ds(j1 * 128, 128)],
                            den_sh.at[dst_v.at[j1]], add=True)
        return 0
    lax.fori_loop(0, nb // 2, body, 0)


def _feat_pass(nb, rowtab_hbm, src_v, dst_v, w_v, acc_sh, rows_v, semA, semB):
    def fire(j, rbase, sem):
        pltpu.async_copy(rowtab_hbm.at[src_v.at[pl.ds(j * 128, 128)]],
                         rows_v.at[pl.ds(rbase, 128)], sem)

    def wait(rbase, sem):
        pltpu.make_async_copy(rowtab_hbm.at[src_v.at[pl.ds(0, 128)]],
                              rows_v.at[pl.ds(rbase, 128)], sem).wait()

    fire(0, 0, semA)

    def body(t, _):
        j0 = 2 * t
        j1 = 2 * t + 1
        fire(j1, 128, semB)
        wait(0, semA)
        _scale_rows(rows_v, 0, w_v, j0 * 128)
        pltpu.sync_copy(rows_v.at[pl.ds(0, 128)],
                        acc_sh.at[dst_v.at[j0]], add=True)

        @pl.when(t < nb // 2 - 1)
        def _():
            fire(j0 + 2, 0, semA)
        wait(128, semB)
        _scale_rows(rows_v, 128, w_v, j1 * 128)
        pltpu.sync_copy(rows_v.at[pl.ds(128, 128)],
                        acc_sh.at[dst_v.at[j1]], add=True)
        return 0
    lax.fori_loop(0, nb // 2, body, 0)


_SC1_SCRATCH = [pltpu.VMEM((WIN * 128,), jnp.int32),
                pltpu.VMEM((WIN, 128), jnp.int32),
                pltpu.VMEM((WIN * 128,), F32),
                pltpu.VMEM((256, 128), F32),
                pltpu.VMEM((128,), F32),
                pltpu.VMEM((128,), F32),
                pltpu.VMEM((128,), F32),
                pltpu.VMEM((128,), F32),
                pltpu.VMEM_SHARED((ACC_R, 128), F32),
                pltpu.VMEM_SHARED((NPAD,), F32),
                pltpu.SemaphoreType.DMA,
                pltpu.SemaphoreType.DMA]


def _den_zero_stripe(den_sh, asbA, sid):
    z = jnp.zeros((16,), F32)
    for i in range(0, 128, 16):
        asbA[pl.ds(i, 16)] = z
    for kk in range(5):
        pltpu.sync_copy(asbA, den_sh.at[pl.ds(sid * DSTR + kk * 128, 128)])


def _den_dump_stripe(den_sh, asbA, den_out, base, sid):
    for kk in range(5):
        pltpu.sync_copy(den_sh.at[pl.ds(sid * DSTR + kk * 128, 128)], asbA)
        pltpu.sync_copy(asbA, den_out.at[pl.ds(base + sid * DSTR
                                               + kk * 128, 128)])


def _edge_windows(s_idx, scale8, do_den, nwin, nb, bbase, stab, dtab,
                  rowtab_hbm, src_hbm, dst2d_hbm, src_v, dst_v, w_v,
                  rows_v, acc_sh, den_sh, asbA, adbA, asbB, adbB,
                  semA, semB):
    def win_body(win, _):
        wb = bbase + win * WIN
        wcount = jnp.minimum(nb - win * WIN, WIN)
        pltpu.sync_copy(src_hbm.at[pl.ds(wb * 128, WIN * 128)], src_v)
        pltpu.sync_copy(dst2d_hbm.at[pl.ds(wb, WIN)], dst_v)
        _w_pass(wcount, stab, dtab, src_v, dst_v, w_v, den_sh,
                wb * 128, do_den, asbA, adbA, asbB, adbB, semA, semB)
        if scale8 is not None:
            def sc_idx(j, _):
                for c in range(8):
                    off = j * 128 + c * 16
                    src_v[pl.ds(off, 16)] = (src_v[pl.ds(off, 16)] * 8
                                             + scale8)
                return 0
            lax.fori_loop(0, wcount, sc_idx, 0)
        _feat_pass(wcount, rowtab_hbm, src_v, dst_v, w_v, acc_sh, rows_v,
                   semA, semB)
        return 0
    lax.fori_loop(0, nwin, win_body, 0)


def _sc_layer1(src, dst2d, atabs, h1r):
    @functools.partial(
        pl.kernel, mesh=_mesh(),
        out_type=[jax.ShapeDtypeStruct((8, NC, ACC_R, 128), F32),
                  jax.ShapeDtypeStruct((NC * 4 * NPAD,), F32)],
        scratch_types=_SC1_SCRATCH,
        compiler_params=pltpu.CompilerParams(needs_layout_passes=False))
    def k(src_hbm, dst2d_hbm, as0, as1, as2, as3, ad0, ad1, ad2, ad3,
          h1r_hbm, acc_out, den_out,
          src_v, dst_v, w_v, rows_v, asbA, adbA, asbB, adbB,
          acc_sh, den_sh, semA, semB):
        cid = lax.axis_index("c")
        sid = lax.axis_index("s")
        nb = lax.select(cid == 0, B0, B1)
        nwin = lax.select(cid == 0, NW0, NW1)
        bbase = lax.select(cid == 0, sid * B0, NS * B0 + sid * B1)
        stabs = [as0, as1, as2, as3]
        dtabs = [ad0, ad1, ad2, ad3]

        _den_zero_stripe(den_sh, asbA, sid)

        for s in range(8):
            h = s // 2
            _zero_rows(rows_v, 128)
            _zero_acc_stripe(acc_sh, rows_v, sid)
            plsc.subcore_barrier()

            _edge_windows(s, s, s % 2 == 0, nwin, nb, bbase,
                          stabs[h], dtabs[h], h1r_hbm, src_hbm, dst2d_hbm,
                          src_v, dst_v, w_v, rows_v, acc_sh, den_sh,
                          asbA, adbA, asbB, adbB, semA, semB)
            plsc.subcore_barrier()

            _dump_acc_stripe(acc_sh, rows_v, acc_out.at[s, cid], sid)
            if s % 2 == 1:
                _den_dump_stripe(den_sh, asbA, den_out,
                                 cid * 4 * NPAD + h * NPAD, sid)
                _den_zero_stripe(den_sh, asbA, sid)

    return k(src, dst2d, *atabs, h1r)


def _sc_layer2(src, dst2d, stab, dtab, h2):
    @functools.partial(
        pl.kernel, mesh=_mesh(),
        out_type=[jax.ShapeDtypeStruct((NC, ACC_R, 128), F32),
                  jax.ShapeDtypeStruct((NC * NPAD,), F32)],
        scratch_types=_SC1_SCRATCH,
        compiler_params=pltpu.CompilerParams(needs_layout_passes=False))
    def k(src_hbm, dst2d_hbm, stab_hbm, dtab_hbm, h2_hbm, acc_out, den_out,
          src_v, dst_v, w_v, rows_v, asbA, adbA, asbB, adbB,
          acc_sh, den_sh, semA, semB):
        cid = lax.axis_index("c")
        sid = lax.axis_index("s")
        nb = lax.select(cid == 0, B0, B1)
        nwin = lax.select(cid == 0, NW0, NW1)
        bbase = lax.select(cid == 0, sid * B0, NS * B0 + sid * B1)

        _den_zero_stripe(den_sh, asbA, sid)
        _zero_rows(rows_v, 128)
        _zero_acc_stripe(acc_sh, rows_v, sid)
        plsc.subcore_barrier()

        _edge_windows(0, None, True, nwin, nb, bbase,
                      stab_hbm, dtab_hbm, h2_hbm, src_hbm, dst2d_hbm,
                      src_v, dst_v, w_v, rows_v, acc_sh, den_sh,
                      asbA, adbA, asbB, adbB, semA, semB)
        plsc.subcore_barrier()

        _den_dump_stripe(den_sh, asbA, den_out, cid * NPAD, sid)
        _dump_acc_stripe(acc_sh, rows_v, acc_out.at[cid], sid)

    return k(src, dst2d, stab, dtab, h2)


def kernel(x, edge_index, batch, W1, att_src1, att_dst1, bias1,
           W2, att_src2, att_dst2, bias2):
    srcp = jnp.concatenate(
        [edge_index[0], jnp.zeros((EP2 - E,), jnp.int32)])
    dstp = jnp.concatenate(
        [edge_index[1], jnp.zeros((EP2 - E,), jnp.int32)])

    attf = jnp.stack([att_src1.reshape(-1), att_dst1.reshape(-1)])
    h1, a1 = _stage_a(x, W1, attf)

    h1r = h1.reshape(N * 8, 128)
    h1s = h1.reshape(N, 8, 128)
    dst2d = dstp.reshape(EP2 // 128, 128)
    a1p = jnp.pad(a1, ((0, NPAD - N), (0, 0)))
    atabs = [a1p[:, i] for i in range(8)]
    acc1, den1f = _sc_layer1(srcp, dst2d, atabs, h1r)
    den1t = jnp.transpose(den1f.reshape(NC, 4, NPAD)[:, :, :N], (0, 2, 1))

    b1r = bias1.reshape(8, 128)
    W2r = W2.reshape(8, 128, 128)
    att2 = jnp.concatenate([att_src2, att_dst2], axis=0)
    b2 = bias2.reshape(1, 128)
    h2, a2 = _stage_d(acc1, den1t, a1, h1s, b1r, W2r, att2, b2)

    a2p = jnp.pad(a2, ((0, NPAD - N), (0, 0)))
    acc2, den2f = _sc_layer2(srcp, dst2d, a2p[:, 0], a2p[:, 1], h2)
    den2t = den2f.reshape(NC, NPAD)[:, :N].T

    batch2d = batch.reshape(N, 1)
    return _stage_f(acc2, den2t, a2, h2, b2, batch2d)


# speedup vs baseline: 1.1143x; 1.0687x over previous
import functools

import jax
import jax.numpy as jnp
from jax import lax
from jax.experimental import pallas as pl
from jax.experimental.pallas import tpu as pltpu
from jax.experimental.pallas import tpu_sc as plsc

N = 10000
E = 160000
EP = 163840
BN = 1000
GRID = N // BN
F32 = jnp.float32


def _lrelu(x):
    return jnp.where(x >= 0, x, 0.2 * x)


def _stage_a_body(x_ref, w1_ref, attf_ref, h1_ref, a1_ref):
    h = jnp.dot(x_ref[...], w1_ref[...], preferred_element_type=F32)
    h1_ref[...] = h
    asrc = (h * attf_ref[0][None, :]).reshape(BN, 4, 256).sum(axis=-1)
    adst = (h * attf_ref[1][None, :]).reshape(BN, 4, 256).sum(axis=-1)
    a1_ref[...] = jnp.concatenate([asrc, adst], axis=1)


def _stage_a(x, W1, attf):
    return pl.pallas_call(
        _stage_a_body,
        grid=(GRID,),
        in_specs=[
            pl.BlockSpec((BN, 256), lambda i: (i, 0)),
            pl.BlockSpec((256, 1024), lambda i: (0, 0)),
            pl.BlockSpec((2, 1024), lambda i: (0, 0)),
        ],
        out_specs=[
            pl.BlockSpec((BN, 1024), lambda i: (i, 0)),
            pl.BlockSpec((BN, 8), lambda i: (i, 0)),
        ],
        out_shape=[
            jax.ShapeDtypeStruct((N, 1024), F32),
            jax.ShapeDtypeStruct((N, 8), F32),
        ],
    )(x, W1, attf)


def _stage_d_body(acc_ref, den_ref, a1_ref, h1s_ref, b1r_ref, w2r_ref,
                  att2_ref, b2_ref, h2_ref, a2_ref):
    den = den_ref[0] + den_ref[1]
    h2 = jnp.zeros((BN, 128), F32)
    for s in range(8):
        hh = s // 2
        wself = jnp.exp(_lrelu(a1_ref[:, hh:hh + 1] + a1_ref[:, 4 + hh:5 + hh]))
        t = acc_ref[s, 0] + acc_ref[s, 1] + wself * h1s_ref[:, s, :]
        t = t / (den[:, hh:hh + 1] + wself)
        t = t + b1r_ref[s][None, :]
        e = jnp.where(t > 0, t, jnp.exp(t) - 1.0)
        h2 = h2 + jnp.dot(e, w2r_ref[s], preferred_element_type=F32)
    h2 = h2 + b2_ref[...]
    h2_ref[...] = h2
    a2s = jnp.sum(h2 * att2_ref[0][None, :], axis=-1, keepdims=True)
    a2d = jnp.sum(h2 * att2_ref[1][None, :], axis=-1, keepdims=True)
    a2_ref[...] = jnp.concatenate([a2s, a2d], axis=1)


def _stage_d(acc1, den1t, a1, h1s, b1r, W2r, att2, b2):
    return pl.pallas_call(
        _stage_d_body,
        grid=(GRID,),
        in_specs=[
            pl.BlockSpec((8, 2, BN, 128), lambda i: (0, 0, i, 0)),
            pl.BlockSpec((2, BN, 4), lambda i: (0, i, 0)),
            pl.BlockSpec((BN, 8), lambda i: (i, 0)),
            pl.BlockSpec((BN, 8, 128), lambda i: (i, 0, 0)),
            pl.BlockSpec((8, 128), lambda i: (0, 0)),
            pl.BlockSpec((8, 128, 128), lambda i: (0, 0, 0)),
            pl.BlockSpec((2, 128), lambda i: (0, 0)),
            pl.BlockSpec((1, 128), lambda i: (0, 0)),
        ],
        out_specs=[
            pl.BlockSpec((BN, 128), lambda i: (i, 0)),
            pl.BlockSpec((BN, 2), lambda i: (i, 0)),
        ],
        out_shape=[
            jax.ShapeDtypeStruct((N, 128), F32),
            jax.ShapeDtypeStruct((N, 2), F32),
        ],
    )(acc1, den1t, a1, h1s, b1r, W2r, att2, b2)


def _stage_f_body(acc2_ref, den2_ref, a2_ref, h2_ref, b2_ref, batch_ref,
                  out_ref, cnt_ref):
    i = pl.program_id(0)
    wself = jnp.exp(_lrelu(a2_ref[:, 0:1] + a2_ref[:, 1:2]))
    den = den2_ref[:, 0:1] + den2_ref[:, 1:2] + wself
    t = (acc2_ref[0] + acc2_ref[1] + wself * h2_ref[...]) / den + b2_ref[...]
    gids = lax.broadcasted_iota(jnp.int32, (BN, 64), 1)
    oh = (batch_ref[...] == gids).astype(F32)
    pool = lax.dot_general(oh, t, (((0,), (0,)), ((), ())),
                           preferred_element_type=F32)
    c128 = lax.dot_general(oh, jnp.ones((BN, 128), F32),
                           (((0,), (0,)), ((), ())),
                           preferred_element_type=F32)

    @pl.when(i == 0)
    def _():
        out_ref[...] = jnp.zeros_like(out_ref)
        cnt_ref[...] = jnp.zeros_like(cnt_ref)

    out_ref[...] += pool
    cnt_ref[...] += c128

    @pl.when(i == GRID - 1)
    def _():
        out_ref[...] = out_ref[...] / jnp.maximum(cnt_ref[...], 1.0)


def _stage_f(acc2, den2t, a2, h2, b2, batch2d):
    return pl.pallas_call(
        _stage_f_body,
        grid=(GRID,),
        in_specs=[
            pl.BlockSpec((2, BN, 128), lambda i: (0, i, 0)),
            pl.BlockSpec((BN, 2), lambda i: (i, 0)),
            pl.BlockSpec((BN, 2), lambda i: (i, 0)),
            pl.BlockSpec((BN, 128), lambda i: (i, 0)),
            pl.BlockSpec((1, 128), lambda i: (0, 0)),
            pl.BlockSpec((BN, 1), lambda i: (i, 0)),
        ],
        out_specs=pl.BlockSpec((64, 128), lambda i: (0, 0)),
        out_shape=jax.ShapeDtypeStruct((64, 128), F32),
        scratch_shapes=[pltpu.VMEM((64, 128), F32)],
    )(acc2, den2t, a2, h2, b2, batch2d)


NC = 2
NS = 16
NW = NC * NS
EW = EP // NW
NB = EW // 128
B0 = 72
B1 = 80 - B0
WIN = 24
NW0 = (B0 + WIN - 1) // WIN
NW1 = (B1 + WIN - 1) // WIN
EP2 = EP + WIN * 128
NPAD = 10240
ACC_R = 10112
STR = ACC_R // NS
DSTR = NPAD // NS
CHUNKS = [(0, 128), (128, 128), (256, 128), (384, 128), (512, 120)]

_sc_mesh = None


def _mesh():
    global _sc_mesh
    if _sc_mesh is None:
        _sc_mesh = plsc.VectorSubcoreMesh(
            core_axis_name="c", subcore_axis_name="s",
            num_cores=NC, num_subcores=NS)
    return _sc_mesh


def _splat(val):
    return lax.iota(jnp.int32, 16) * 0 + val


def _iota16():
    return lax.iota(jnp.int32, 16)


def _zero_vec(ref, n):
    z = jnp.zeros((16,), F32)
    for i in range(0, n, 16):
        ref[pl.ds(i, 16)] = z


def _zero_rows(rows_v, n):
    z = jnp.zeros((16,), F32)

    def body(r, _):
        for c in range(8):
            rows_v[r, pl.ds(c * 16, 16)] = z
        return 0
    lax.fori_loop(0, n, body, 0)


def _zero_acc_stripe(acc_sh, rows_v, sid):
    for off, ln in CHUNKS:
        pltpu.sync_copy(rows_v.at[pl.ds(0, ln)],
                        acc_sh.at[pl.ds(sid * STR + off, ln)])


def _dump_acc_stripe(acc_sh, rows_v, out_at, sid):
    for off, ln in CHUNKS:
        pltpu.sync_copy(acc_sh.at[pl.ds(sid * STR + off, ln)],
                        rows_v.at[pl.ds(0, ln)])
        pltpu.sync_copy(rows_v.at[pl.ds(0, ln)],
                        out_at.at[pl.ds(sid * STR + off, ln)])


def _scale_rows(rows_v, rbase, w_v, wbase):
    def body(r, _):
        wsp = plsc.load_gather(w_v, [_splat(wbase + r)])
        for c in range(8):
            rows_v[rbase + r, pl.ds(c * 16, 16)] = (
                rows_v[rbase + r, pl.ds(c * 16, 16)] * wsp)
        return 0
    lax.fori_loop(0, 128, body, 0)


def _fire_alpha(stab, dtab, src_v, dst_v, j, asb, adb, sem):
    pltpu.async_copy(stab.at[src_v.at[pl.ds(j * 128, 128)]], asb, sem)
    pltpu.async_copy(dtab.at[dst_v.at[j]], adb, sem)


def _wait_alpha(stab, dtab, src_v, dst_v, asb, adb, sem):
    pltpu.make_async_copy(stab.at[src_v.at[pl.ds(0, 128)]], asb, sem).wait()
    pltpu.make_async_copy(dtab.at[dst_v.at[0]], adb, sem).wait()


def _w_block(j, asb, adb, w_v, ebase):
    for c in range(8):
        a = asb[pl.ds(c * 16, 16)] + adb[pl.ds(c * 16, 16)]
        a = jnp.where(a >= 0, a, a * 0.2)
        w16 = jnp.exp(a)
        gid = _iota16() + (ebase + j * 128 + c * 16)
        w_v[pl.ds(j * 128 + c * 16, 16)] = jnp.where(gid < E, w16, 0.0)


def _w_pass(nb, stab, dtab, src_v, dst_v, w_v, den_sh, ebase, do_den,
            asbA, adbA, asbB, adbB, semA, semB):
    _fire_alpha(stab, dtab, src_v, dst_v, 0, asbA, adbA, semA)

    def body(t, _):
        j0 = 2 * t
        j1 = 2 * t + 1
        _fire_alpha(stab, dtab, src_v, dst_v, j1, asbB, adbB, semB)
        _wait_alpha(stab, dtab, src_v, dst_v, asbA, adbA, semA)
        _w_block(j0, asbA, adbA, w_v, ebase)
        if do_den:
            pltpu.sync_copy(w_v.at[pl.ds(j0 * 128, 128)],
                            den_sh.at[dst_v.at[j0]], add=True)

        @pl.when(t < nb // 2 - 1)
        def _():
            _fire_alpha(stab, dtab, src_v, dst_v, j0 + 2, asbA, adbA, semA)
        _wait_alpha(stab, dtab, src_v, dst_v, asbB, adbB, semB)
        _w_block(j1, asbB, adbB, w_v, ebase)
        if do_den:
            pltpu.sync_copy(w_v.at[pl.ds(j1 * 128, 128)],
                            den_sh.at[dst_v.at[j1]], add=True)
        return 0
    lax.fori_loop(0, nb // 2, body, 0)


def _feat_pass(nb, rowtab_hbm, src_v, dst_v, w_v, acc_sh, rows_v, semA, semB):
    def fire(j, rbase, sem):
        pltpu.async_copy(rowtab_hbm.at[src_v.at[pl.ds(j * 128, 128)]],
                         rows_v.at[pl.ds(rbase, 128)], sem)

    def wait(rbase, sem):
        pltpu.make_async_copy(rowtab_hbm.at[src_v.at[pl.ds(0, 128)]],
                              rows_v.at[pl.ds(rbase, 128)], sem).wait()

    fire(0, 0, semA)

    def body(t, _):
        j0 = 2 * t
        j1 = 2 * t + 1
        fire(j1, 128, semB)
        wait(0, semA)
        _scale_rows(rows_v, 0, w_v, j0 * 128)
        pltpu.sync_copy(rows_v.at[pl.ds(0, 128)],
                        acc_sh.at[dst_v.at[j0]], add=True)

        @pl.when(t < nb // 2 - 1)
        def _():
            fire(j0 + 2, 0, semA)
        wait(128, semB)
        _scale_rows(rows_v, 128, w_v, j1 * 128)
        pltpu.sync_copy(rows_v.at[pl.ds(128, 128)],
                        acc_sh.at[dst_v.at[j1]], add=True)
        return 0
    lax.fori_loop(0, nb // 2, body, 0)


_SC1_SCRATCH = [pltpu.VMEM((WIN * 128,), jnp.int32),
                pltpu.VMEM((WIN, 128), jnp.int32),
                pltpu.VMEM((WIN * 128,), F32),
                pltpu.VMEM((256, 128), F32),
                pltpu.VMEM((128,), F32),
                pltpu.VMEM((128,), F32),
                pltpu.VMEM((128,), F32),
                pltpu.VMEM((128,), F32),
                pltpu.VMEM_SHARED((ACC_R, 128), F32),
                pltpu.VMEM_SHARED((NPAD,), F32),
                pltpu.SemaphoreType.DMA,
                pltpu.SemaphoreType.DMA]


def _den_zero_stripe(den_sh, asbA, sid):
    z = jnp.zeros((16,), F32)
    for i in range(0, 128, 16):
        asbA[pl.ds(i, 16)] = z
    for kk in range(5):
        pltpu.sync_copy(asbA, den_sh.at[pl.ds(sid * DSTR + kk * 128, 128)])


def _den_dump_stripe(den_sh, asbA, den_out, base, sid):
    for kk in range(5):
        pltpu.sync_copy(den_sh.at[pl.ds(sid * DSTR + kk * 128, 128)], asbA)
        pltpu.sync_copy(asbA, den_out.at[pl.ds(base + sid * DSTR
                                               + kk * 128, 128)])


def _edge_windows(s_idx, scale8, do_den, nwin, nb, bbase, stab, dtab,
                  rowtab_hbm, src_hbm, dst2d_hbm, src_v, dst_v, w_v,
                  rows_v, acc_sh, den_sh, asbA, adbA, asbB, adbB,
                  semA, semB):
    def win_body(win, _):
        wb = bbase + win * WIN
        wcount = jnp.minimum(nb - win * WIN, WIN)
        pltpu.sync_copy(src_hbm.at[pl.ds(wb * 128, WIN * 128)], src_v)
        pltpu.sync_copy(dst2d_hbm.at[pl.ds(wb, WIN)], dst_v)
        _w_pass(wcount, stab, dtab, src_v, dst_v, w_v, den_sh,
                wb * 128, do_den, asbA, adbA, asbB, adbB, semA, semB)
        if scale8 is not None:
            def sc_idx(j, _):
                for c in range(8):
                    off = j * 128 + c * 16
                    src_v[pl.ds(off, 16)] = (src_v[pl.ds(off, 16)] * 8
                                             + scale8)
                return 0
            lax.fori_loop(0, wcount, sc_idx, 0)
        _feat_pass(wcount, rowtab_hbm, src_v, dst_v, w_v, acc_sh, rows_v,
                   semA, semB)
        return 0
    lax.fori_loop(0, nwin, win_body, 0)


def _sc_layer1(src, dst2d, atabs, h1r):
    @functools.partial(
        pl.kernel, mesh=_mesh(),
        out_type=[jax.ShapeDtypeStruct((8, NC, ACC_R, 128), F32),
                  jax.ShapeDtypeStruct((NC * 4 * NPAD,), F32)],
        scratch_types=_SC1_SCRATCH,
        compiler_params=pltpu.CompilerParams(needs_layout_passes=False))
    def k(src_hbm, dst2d_hbm, as0, as1, as2, as3, ad0, ad1, ad2, ad3,
          h1r_hbm, acc_out, den_out,
          src_v, dst_v, w_v, rows_v, asbA, adbA, asbB, adbB,
          acc_sh, den_sh, semA, semB):
        cid = lax.axis_index("c")
        sid = lax.axis_index("s")
        nb = lax.select(cid == 0, B0, B1)
        nwin = lax.select(cid == 0, NW0, NW1)
        bbase = lax.select(cid == 0, sid * B0, NS * B0 + sid * B1)
        stabs = [as0, as1, as2, as3]
        dtabs = [ad0, ad1, ad2, ad3]

        _den_zero_stripe(den_sh, asbA, sid)

        for s in range(8):
            h = s // 2
            _zero_rows(rows_v, 128)
            _zero_acc_stripe(acc_sh, rows_v, sid)
            plsc.subcore_barrier()

            _edge_windows(s, s, s % 2 == 0, nwin, nb, bbase,
                          stabs[h], dtabs[h], h1r_hbm, src_hbm, dst2d_hbm,
                          src_v, dst_v, w_v, rows_v, acc_sh, den_sh,
                          asbA, adbA, asbB, adbB, semA, semB)
            plsc.subcore_barrier()

            _dump_acc_stripe(acc_sh, rows_v, acc_out.at[s, cid], sid)
            if s % 2 == 1:
                _den_dump_stripe(den_sh, asbA, den_out,
                                 cid * 4 * NPAD + h * NPAD, sid)
                _den_zero_stripe(den_sh, asbA, sid)

    return k(src, dst2d, *atabs, h1r)


def _sc_layer2(src, dst2d, stab, dtab, h2):
    @functools.partial(
        pl.kernel, mesh=_mesh(),
        out_type=[jax.ShapeDtypeStruct((NC, ACC_R, 128), F32),
                  jax.ShapeDtypeStruct((NC * NPAD,), F32)],
        scratch_types=_SC1_SCRATCH,
        compiler_params=pltpu.CompilerParams(needs_layout_passes=False))
    def k(src_hbm, dst2d_hbm, stab_hbm, dtab_hbm, h2_hbm, acc_out, den_out,
          src_v, dst_v, w_v, rows_v, asbA, adbA, asbB, adbB,
          acc_sh, den_sh, semA, semB):
        cid = lax.axis_index("c")
        sid = lax.axis_index("s")
        nb = lax.select(cid == 0, B0, B1)
        nwin = lax.select(cid == 0, NW0, NW1)
        bbase = lax.select(cid == 0, sid * B0, NS * B0 + sid * B1)

        _den_zero_stripe(den_sh, asbA, sid)
        _zero_rows(rows_v, 128)
        _zero_acc_stripe(acc_sh, rows_v, sid)
        plsc.subcore_barrier()

        _edge_windows(0, None, True, nwin, nb, bbase,
                      stab_hbm, dtab_hbm, h2_hbm, src_hbm, dst2d_hbm,
                      src_v, dst_v, w_v, rows_v, acc_sh, den_sh,
                      asbA, adbA, asbB, adbB, semA, semB)
        plsc.subcore_barrier()

        _den_dump_stripe(den_sh, asbA, den_out, cid * NPAD, sid)
        _dump_acc_stripe(acc_sh, rows_v, acc_out.at[cid], sid)

    return k(src, dst2d, stab, dtab, h2)


def kernel(x, edge_index, batch, W1, att_src1, att_dst1, bias1,
           W2, att_src2, att_dst2, bias2):
    srcp = jnp.concatenate(
        [edge_index[0], jnp.zeros((EP2 - E,), jnp.int32)])
    dstp = jnp.concatenate(
        [edge_index[1], jnp.zeros((EP2 - E,), jnp.int32)])

    attf = jnp.stack([att_src1.reshape(-1), att_dst1.reshape(-1)])
    h1, a1 = _stage_a(x, W1, attf)

    h1r = h1.reshape(N * 8, 128)
    h1s = h1.reshape(N, 8, 128)
    dst2d = dstp.reshape(EP2 // 128, 128)
    a1p = jnp.pad(a1, ((0, NPAD - N), (0, 0)))
    atabs = [a1p[:, i] for i in range(8)]
    acc1, den1f = _sc_layer1(srcp, dst2d, atabs, h1r)
    den1t = jnp.transpose(den1f.reshape(NC, 4, NPAD)[:, :, :N], (0, 2, 1))

    b1r = bias1.reshape(8, 128)
    W2r = W2.reshape(8, 128, 128)
    att2 = jnp.concatenate([att_src2, att_dst2], axis=0)
    b2 = bias2.reshape(1, 128)
    h2, a2 = _stage_d(acc1, den1t, a1, h1s, b1r, W2r, att2, b2)

    a2p = jnp.pad(a2, ((0, NPAD - N), (0, 0)))
    acc2, den2f = _sc_layer2(srcp, dst2d, a2p[:, 0], a2p[:, 1], h2)
    den2t = den2f.reshape(NC, NPAD)[:, :N].T

    batch2d = batch.reshape(N, 1)
    return _stage_f(acc2, den2t, a2, h2, b2, batch2d)
